# baseline (speedup 1.0000x reference)
import functools

import jax
import jax.numpy as jnp
import numpy as np
from jax import lax
from jax.experimental import pallas as pl
from jax.experimental.pallas import tpu as pltpu
from jax.experimental.pallas import tpu_sc as plsc

_TINY = np.float32(np.finfo(np.float32).tiny)


def _threefry2x32(k0, k1, x0, x1):
    k2 = k0 ^ k1 ^ np.uint32(0x1BD11BDA)
    ks = (k0, k1, k2)
    rot = ((13, 15, 26, 6), (17, 29, 16, 24))
    x0 = x0 + k0
    x1 = x1 + k1
    for i in range(5):
        for d in rot[i % 2]:
            x0 = x0 + x1
            x1 = (x1 << np.uint32(d)) | (x1 >> np.uint32(32 - d))
            x1 = x1 ^ x0
        x0 = x0 + ks[(i + 1) % 3]
        x1 = x1 + ks[(i + 2) % 3] + np.uint32(i + 1)
    return x0, x1


def _exp_from_bits(bits):
    fb = (bits >> np.uint32(9)) | np.uint32(0x3F800000)
    u = jax.lax.bitcast_convert_type(fb, jnp.float32) - jnp.float32(1.0)
    return -jnp.log1p(-u)


def _scan_body(V, C, key_ref, t_ref, d_ref, idx_out, run_max, run_idx, pre):
    j = pl.program_id(0)
    R = t_ref.shape[0]

    @pl.when(j == 0)
    def _init():
        run_max[...] = jnp.full_like(run_max, -jnp.inf)
        run_idx[...] = jnp.zeros_like(run_idx)
        rows = jax.lax.broadcasted_iota(jnp.int32, (R, C), 0)
        li = jax.lax.broadcasted_iota(jnp.int32, (R, C), 1)
        pre[...] = (rows * V + li).astype(jnp.uint32)

    cols = jax.lax.broadcasted_iota(jnp.int32, (R, C), 1) + j * C
    valid = cols < V
    t = t_ref[...]
    d = d_ref[...]

    x1 = pre[...] + (j * C).astype(jnp.uint32)
    o0, o1 = _threefry2x32(key_ref[0], key_ref[1], np.uint32(0), x1)
    q = _exp_from_bits(o0 ^ o1)

    f = jnp.maximum(t - d, _TINY)
    s = jnp.where(valid, f / q, jnp.float32(-1.0))

    cmax = jnp.max(s, axis=1, keepdims=True)
    carg = jnp.min(jnp.where(s == cmax, cols, jnp.int32(2**30)),
                   axis=1, keepdims=True)

    upd = cmax > run_max[:, 0:1]
    run_max[:, 0:1] = jnp.where(upd, cmax, run_max[:, 0:1])
    run_idx[:, 0:1] = jnp.where(upd, carg, run_idx[:, 0:1])

    @pl.when(j == pl.num_programs(0) - 1)
    def _fin():
        idx_out[...] = run_idx[...]


def _sc_gather_body(t_hbm, d_hbm, idx_hbm, selt_hbm, seld_hbm,
                    idx_v, tv, dv, sem):
    wid = lax.axis_index("s") * 2 + lax.axis_index("c")
    base = wid * 16
    pltpu.sync_copy(idx_hbm.at[pl.ds(base, 16)], idx_v)
    pltpu.async_copy(t_hbm.at[idx_v], tv, sem).wait()
    pltpu.async_copy(d_hbm.at[idx_v], dv, sem).wait()
    pltpu.sync_copy(tv, selt_hbm.at[pl.ds(base, 16)])
    pltpu.sync_copy(dv, seld_hbm.at[pl.ds(base, 16)])


def _out_body(B, K, u_ref, selt_ref, seld_ref, draft9_ref, rec9_ref,
              bonus_ref, out_ref):
    u = u_ref[...]
    ratio = jnp.minimum(selt_ref[...] / seld_ref[...], jnp.float32(1.0))
    rej = jnp.logical_not(u < ratio)
    kidx = jax.lax.broadcasted_iota(jnp.int32, (B, K), 1)
    limit = jnp.min(jnp.where(rej, kidx, jnp.int32(K)),
                    axis=1, keepdims=True)

    k9 = jax.lax.broadcasted_iota(jnp.int32, (B, K + 1), 1)
    draft9 = draft9_ref[...]
    rec9 = rec9_ref[...]
    bonus = jnp.broadcast_to(bonus_ref[...], (B, K + 1))
    neg1 = jnp.full((B, K + 1), -1, jnp.int32)

    inner = jnp.where(k9 < limit, draft9,
                      jnp.where(k9 == limit, rec9, neg1))
    out_ref[...] = jnp.where(k9 == K,
                             jnp.where(limit == K, bonus, neg1),
                             inner)


def kernel(target_probs, bonus_token_ids, draft_probs, draft_token_ids):
    B, K, V = target_probs.shape
    R = B * K

    rkey = jax.random.key(1)
    ku, kq = jax.random.split(rkey)
    u = jax.random.uniform(ku, (B, K), dtype=jnp.float32)
    kq_data = jax.random.key_data(kq)

    t2 = target_probs.reshape(R, V)
    d2 = draft_probs.reshape(R, V)

    abs_idx = jnp.arange(R, dtype=jnp.int32) * V + draft_token_ids.reshape(R)
    sc_gather = pl.kernel(
        _sc_gather_body,
        out_type=[
            jax.ShapeDtypeStruct((R,), jnp.float32),
            jax.ShapeDtypeStruct((R,), jnp.float32),
        ],
        mesh=plsc.VectorSubcoreMesh(core_axis_name="c", subcore_axis_name="s"),
        scratch_types=[
            pltpu.VMEM((16,), jnp.int32),
            pltpu.VMEM((16,), jnp.float32),
            pltpu.VMEM((16,), jnp.float32),
            pltpu.SemaphoreType.DMA,
        ],
    )
    selt_flat, seld_flat = sc_gather(t2.reshape(-1), d2.reshape(-1), abs_idx)

    C = 2048 if V >= 2048 else ((V + 127) // 128) * 128
    nchunks = (V + C - 1) // C

    idx = pl.pallas_call(
        functools.partial(_scan_body, V, C),
        grid=(nchunks,),
        in_specs=[
            pl.BlockSpec(memory_space=pltpu.SMEM),
            pl.BlockSpec((R, C), lambda j: (0, j)),
            pl.BlockSpec((R, C), lambda j: (0, j)),
        ],
        out_specs=pl.BlockSpec((R, 128), lambda j: (0, 0)),
        out_shape=jax.ShapeDtypeStruct((R, 128), jnp.int32),
        scratch_shapes=[
            pltpu.VMEM((R, 128), jnp.float32),
            pltpu.VMEM((R, 128), jnp.int32),
            pltpu.VMEM((R, C), jnp.uint32),
        ],
    )(kq_data, t2, d2)

    rec = idx[:, 0].reshape(B, K)
    sel_t = selt_flat.reshape(B, K)
    sel_d = seld_flat.reshape(B, K)

    pad = ((0, 0), (0, 1))
    draft9 = jnp.pad(draft_token_ids, pad)
    rec9 = jnp.pad(rec, pad)

    out = pl.pallas_call(
        functools.partial(_out_body, B, K),
        out_shape=jax.ShapeDtypeStruct((B, K + 1), jnp.int32),
    )(u, sel_t, sel_d, draft9, rec9, bonus_token_ids)
    return out


# speedup vs baseline: 1.9749x; 1.9749x over previous
import functools

import jax
import jax.numpy as jnp
import numpy as np
from jax import lax
from jax.experimental import pallas as pl
from jax.experimental.pallas import tpu as pltpu
from jax.experimental.pallas import tpu_sc as plsc

_TINY = np.float32(np.finfo(np.float32).tiny)

_KQ = np.asarray(
    jax.random.key_data(jax.random.split(jax.random.key(1))[1]),
    dtype=np.uint32)


def _threefry2x32(k0, k1, x0, x1):
    k2 = k0 ^ k1 ^ np.uint32(0x1BD11BDA)
    ks = (k0, k1, k2)
    rot = ((13, 15, 26, 6), (17, 29, 16, 24))
    x0 = x0 + k0
    x1 = x1 + k1
    for i in range(5):
        for d in rot[i % 2]:
            x0 = x0 + x1
            x1 = (x1 << np.uint32(d)) | (x1 >> np.uint32(32 - d))
            x1 = x1 ^ x0
        x0 = x0 + ks[(i + 1) % 3]
        x1 = x1 + ks[(i + 2) % 3] + np.uint32(i + 1)
    return x0, x1


def _u_from_bits(bits):
    fb = (bits >> np.uint32(9)) | np.uint32(0x3F800000)
    return jax.lax.bitcast_convert_type(fb, jnp.float32) - jnp.float32(1.0)


def _scan_common(j0, cols, t, d, q, valid, tok_ref, run_max, run_idx,
                 acc_t, acc_d):
    f = jnp.maximum(t - d, _TINY)
    s = f / q
    if valid is not None:
        s = jnp.where(valid, s, jnp.float32(-1.0))

    cmax = jnp.max(s, axis=1, keepdims=True)
    carg = jnp.min(jnp.where(s == cmax, cols, jnp.int32(2**30)),
                   axis=1, keepdims=True)

    tok = tok_ref[:, 0:1]
    m = cols == tok
    st = jnp.sum(jnp.where(m, t, 0.0), axis=1, keepdims=True)
    sd = jnp.sum(jnp.where(m, d, 0.0), axis=1, keepdims=True)

    upd = cmax > run_max[:, 0:1]
    run_max[:, 0:1] = jnp.where(upd, cmax, run_max[:, 0:1])
    run_idx[:, 0:1] = jnp.where(upd, carg, run_idx[:, 0:1])
    acc_t[:, 0:1] = acc_t[:, 0:1] + st
    acc_d[:, 0:1] = acc_d[:, 0:1] + sd


def _scan_a_body(V, C, MASK_A, tok_ref, key_ref, t_ref, d_ref,
                 pm_out, pi_out, pt_out, pd_out,
                 run_max, run_idx, acc_t, acc_d):
    j = pl.program_id(0)
    R = t_ref.shape[0]

    @pl.when(j == 0)
    def _init():
        run_max[...] = jnp.full_like(run_max, -jnp.inf)
        run_idx[...] = jnp.zeros_like(run_idx)
        acc_t[...] = jnp.zeros_like(acc_t)
        acc_d[...] = jnp.zeros_like(acc_d)

    cols = jax.lax.broadcasted_iota(jnp.int32, (R, C), 1) + j * C
    rows = jax.lax.broadcasted_iota(jnp.int32, (R, C), 0)
    x1 = (rows * V + cols).astype(jnp.uint32)
    o0, o1 = _threefry2x32(key_ref[0], key_ref[1], np.uint32(0), x1)
    q = -jnp.log1p(-_u_from_bits(o0 ^ o1))

    valid = (cols < V) if MASK_A else None
    _scan_common(j, cols, t_ref[...], d_ref[...], q, valid, tok_ref,
                 run_max, run_idx, acc_t, acc_d)

    @pl.when(j == pl.num_programs(0) - 1)
    def _fin():
        pm_out[...] = run_max[...]
        pi_out[...] = run_idx[...]
        pt_out[...] = acc_t[...]
        pd_out[...] = acc_d[...]


def _scan_b_body(V, C, SPLIT, tok_ref, t_ref, d_ref, u_ref,
                 pm_ref, pi_ref, pt_ref, pd_ref,
                 idx_out, selt_out, seld_out,
                 run_max, run_idx, acc_t, acc_d):
    j = pl.program_id(0)
    R = t_ref.shape[0]

    @pl.when(j == 0)
    def _init():
        run_max[...] = pm_ref[...]
        run_idx[...] = pi_ref[...]
        acc_t[...] = pt_ref[...]
        acc_d[...] = pd_ref[...]

    cols = (jax.lax.broadcasted_iota(jnp.int32, (R, C), 1)
            + (SPLIT + j) * C)
    valid = cols < V
    q = -jnp.log1p(-u_ref[...])

    _scan_common(j, cols, t_ref[...], d_ref[...], q, valid, tok_ref,
                 run_max, run_idx, acc_t, acc_d)

    @pl.when(j == pl.num_programs(0) - 1)
    def _fin():
        idx_out[...] = run_idx[...]
        selt_out[...] = acc_t[...]
        seld_out[...] = acc_d[...]


def _sc_u_body(V, V0, W, u_hbm, buf, sem):
    wid = lax.axis_index("s") * 2 + lax.axis_index("c")
    row0 = wid * 16
    nvec = W // 16
    lane = lax.iota(jnp.int32, 16)
    k0 = np.uint32(_KQ[0])
    k1 = np.uint32(_KQ[1])

    def row_body(r, carry):
        base = (row0 + r) * V + V0

        def vec_body(k, c2):
            x1 = (base + 16 * k + lane).astype(jnp.uint32)
            o0, o1 = _threefry2x32(k0, k1, np.uint32(0), x1)
            buf[pl.ds(16 * k, 16)] = _u_from_bits(o0 ^ o1)
            return c2

        lax.fori_loop(0, nvec, vec_body, 0, unroll=4)
        pltpu.async_copy(buf, u_hbm.at[row0 + r], sem).wait()
        return carry

    lax.fori_loop(0, 16, row_body, 0)


def _out_body(B, K, u_ref, selt_ref, seld_ref, draft9_ref, rec9_ref,
              bonus_ref, out_ref):
    u = u_ref[...]
    ratio = jnp.minimum(selt_ref[...] / seld_ref[...], jnp.float32(1.0))
    rej = jnp.logical_not(u < ratio)
    kidx = jax.lax.broadcasted_iota(jnp.int32, (B, K), 1)
    limit = jnp.min(jnp.where(rej, kidx, jnp.int32(K)),
                    axis=1, keepdims=True)

    k9 = jax.lax.broadcasted_iota(jnp.int32, (B, K + 1), 1)
    draft9 = draft9_ref[...]
    rec9 = rec9_ref[...]
    bonus = jnp.broadcast_to(bonus_ref[...], (B, K + 1))
    neg1 = jnp.full((B, K + 1), -1, jnp.int32)

    inner = jnp.where(k9 < limit, draft9,
                      jnp.where(k9 == limit, rec9, neg1))
    out_ref[...] = jnp.where(k9 == K,
                             jnp.where(limit == K, bonus, neg1),
                             inner)


def kernel(target_probs, bonus_token_ids, draft_probs, draft_token_ids):
    B, K, V = target_probs.shape
    R = B * K

    rkey = jax.random.key(1)
    ku, kq = jax.random.split(rkey)
    u = jax.random.uniform(ku, (B, K), dtype=jnp.float32)
    kq_data = jax.random.key_data(kq)

    t2 = target_probs.reshape(R, V)
    d2 = draft_probs.reshape(R, V)
    tok_b = jnp.broadcast_to(draft_token_ids.reshape(R, 1), (R, 128))

    C = 2048 if V >= 2048 else ((V + 127) // 128) * 128
    nchunks = (V + C - 1) // C
    SPLIT = max(nchunks - 15, 1) if nchunks > 1 else nchunks
    NSC = nchunks - SPLIT

    part_spec = pl.BlockSpec((R, 128), lambda j: (0, 0))
    part_shape_f = jax.ShapeDtypeStruct((R, 128), jnp.float32)
    part_shape_i = jax.ShapeDtypeStruct((R, 128), jnp.int32)
    scratches = [
        pltpu.VMEM((R, 128), jnp.float32),
        pltpu.VMEM((R, 128), jnp.int32),
        pltpu.VMEM((R, 128), jnp.float32),
        pltpu.VMEM((R, 128), jnp.float32),
    ]

    if NSC > 0:
        W = NSC * C
        V0 = SPLIT * C
        sc_u = pl.kernel(
            functools.partial(_sc_u_body, V, V0, W),
            out_type=jax.ShapeDtypeStruct((R, W), jnp.float32),
            mesh=plsc.VectorSubcoreMesh(core_axis_name="c",
                                        subcore_axis_name="s"),
            scratch_types=[
                pltpu.VMEM((W,), jnp.float32),
                pltpu.SemaphoreType.DMA,
            ],
        )
        u_sc = sc_u()

    pm, pi, pt, pd = pl.pallas_call(
        functools.partial(_scan_a_body, V, C, SPLIT * C > V),
        grid=(SPLIT,),
        in_specs=[
            pl.BlockSpec((R, 128), lambda j: (0, 0)),
            pl.BlockSpec(memory_space=pltpu.SMEM),
            pl.BlockSpec((R, C), lambda j: (0, j)),
            pl.BlockSpec((R, C), lambda j: (0, j)),
        ],
        out_specs=[part_spec] * 4,
        out_shape=[part_shape_f, part_shape_i, part_shape_f, part_shape_f],
        scratch_shapes=scratches,
    )(tok_b, kq_data, t2, d2)

    if NSC > 0:
        idx, selt, seld = pl.pallas_call(
            functools.partial(_scan_b_body, V, C, SPLIT),
            grid=(NSC,),
            in_specs=[
                pl.BlockSpec((R, 128), lambda j: (0, 0)),
                pl.BlockSpec((R, C), lambda j: (0, j + SPLIT)),
                pl.BlockSpec((R, C), lambda j: (0, j + SPLIT)),
                pl.BlockSpec((R, C), lambda j: (0, j)),
                part_spec, part_spec, part_spec, part_spec,
            ],
            out_specs=[part_spec] * 3,
            out_shape=[part_shape_i, part_shape_f, part_shape_f],
            scratch_shapes=scratches,
        )(tok_b, t2, d2, u_sc, pm, pi, pt, pd)
    else:
        idx, selt, seld = pi, pt, pd

    rec = idx[:, 0].reshape(B, K)
    sel_t = selt[:, 0].reshape(B, K)
    sel_d = seld[:, 0].reshape(B, K)

    pad = ((0, 0), (0, 1))
    draft9 = jnp.pad(draft_token_ids, pad)
    rec9 = jnp.pad(rec, pad)

    out = pl.pallas_call(
        functools.partial(_out_body, B, K),
        out_shape=jax.ShapeDtypeStruct((B, K + 1), jnp.int32),
    )(u, sel_t, sel_d, draft9, rec9, bonus_token_ids)
    return out
